# baseline (speedup 1.0000x reference)
import jax
import jax.numpy as jnp
from jax import lax
from jax.experimental import pallas as pl
from jax.experimental.pallas import tpu as pltpu
from jax.experimental.pallas import tpu_sc as plsc

_BETA = 0.66
_GAMMA = -0.1
_ZETA = 1.1

_NC = 2
_NS = 16
_NW = _NC * _NS
_L = 16
_CHUNK = 128
_NPAD = 10240

_SC_PARAMS = pltpu.CompilerParams(
    needs_layout_passes=False, use_tc_tiling_on_sc=False)


def _prologue_call(x, W, b2, attn_l, attn_r, bias_l0):
    n, d = x.shape
    blk = 1000

    def body(x_ref, w_ref, b_ref, al_ref, ar_ref, bias_ref,
             ft_ref, a1_ref, a2_ref):
        ft = lax.dot_general(x_ref[...], w_ref[...], (((1,), (1,)), ((), ())),
                             preferred_element_type=jnp.float32)
        ft = ft + b_ref[...]
        ft_ref[...] = ft
        a1_ref[...] = jnp.sum(ft * al_ref[...], axis=1, keepdims=True)
        a2_ref[...] = jnp.sum(ft * ar_ref[...], axis=1, keepdims=True) + bias_ref[0]

    return pl.pallas_call(
        body,
        grid=(n // blk,),
        in_specs=[
            pl.BlockSpec((blk, d), lambda i: (i, 0)),
            pl.BlockSpec((d, d), lambda i: (0, 0)),
            pl.BlockSpec((1, d), lambda i: (0, 0)),
            pl.BlockSpec((1, d), lambda i: (0, 0)),
            pl.BlockSpec((1, d), lambda i: (0, 0)),
            pl.BlockSpec(memory_space=pltpu.SMEM),
        ],
        out_specs=[
            pl.BlockSpec((blk, d), lambda i: (i, 0)),
            pl.BlockSpec((blk, 1), lambda i: (i, 0)),
            pl.BlockSpec((blk, 1), lambda i: (i, 0)),
        ],
        out_shape=[
            jax.ShapeDtypeStruct((n, d), jnp.float32),
            jax.ShapeDtypeStruct((n, 1), jnp.float32),
            jax.ShapeDtypeStruct((n, 1), jnp.float32),
        ],
    )(x, W, b2, attn_l, attn_r, bias_l0)


def _gate_call(a1, a2, srcp, dstp, cpw, n_edges):
    n = a1.shape[0]
    rows_pt = _NPAD // _NS
    mesh = plsc.VectorSubcoreMesh(core_axis_name="c", subcore_axis_name="s")

    def body(a1_hbm, a2_hbm, src_hbm, dst_hbm, a_hbm, z_hbm, cnt_hbm,
             a1_v, a2_v, src_v, dst_v, a_v, z_v, cnt_v):
        c = lax.axis_index("c")
        s = lax.axis_index("s")
        wid = c * _NS + s

        def zero_z(i, carry):
            z_v[pl.ds(i * _L, _L)] = jnp.zeros((_L,), jnp.float32)
            return carry
        lax.fori_loop(0, _NPAD // _L, zero_z, 0)

        pltpu.sync_copy(a1_hbm, a1_v)
        pltpu.sync_copy(a2_hbm, a2_v)
        pltpu.sync_copy(src_hbm.at[wid], src_v)
        pltpu.sync_copy(dst_hbm.at[wid], dst_v)

        base = wid * (cpw * _CHUNK)
        lanes = lax.iota(jnp.int32, _L)

        def gate_row(r, cnt):
            for cc in range(_CHUNK // _L):
                sv = src_v[r, pl.ds(cc * _L, _L)]
                dv = dst_v[r, pl.ds(cc * _L, _L)]
                a1g = plsc.load_gather(a1_v, [sv])
                a2g = plsc.load_gather(a2_v, [dv])
                logit = a1g + a2g
                sg = 1.0 / (1.0 + jnp.exp(logit * (-1.0 / _BETA)))
                ab = sg * (_ZETA - _GAMMA) + _GAMMA
                a = jnp.clip(ab, 0.0, 1.0)
                a = jnp.where(sv == dv, 1.0, a)
                gid = base + r * _CHUNK + cc * _L + lanes
                a = jnp.where(gid < n_edges, a, 0.0)
                a_v[r, pl.ds(cc * _L, _L)] = a
                plsc.addupdate_scatter(z_v, [dv], a)
                cnt = cnt + jnp.where(a > 0.0, 1, 0).astype(jnp.int32)
            return cnt

        cnt = lax.fori_loop(0, cpw, gate_row, jnp.zeros((_L,), jnp.int32))
        pltpu.sync_copy(a_v, a_hbm.at[wid])
        cnt_v[...] = cnt
        pltpu.sync_copy(cnt_v, cnt_hbm.at[wid, 0])
        pltpu.sync_copy(z_v, z_hbm.at[wid, 0])

    f = pl.kernel(
        body,
        out_type=[
            jax.ShapeDtypeStruct((_NW, cpw, _CHUNK), jnp.float32),
            jax.ShapeDtypeStruct((_NW, 1, _NPAD), jnp.float32),
            jax.ShapeDtypeStruct((_NW, 1, _L), jnp.int32),
        ],
        mesh=mesh,
        compiler_params=_SC_PARAMS,
        scratch_types=[
            pltpu.VMEM((n,), jnp.float32),
            pltpu.VMEM((n,), jnp.float32),
            pltpu.VMEM((cpw, _CHUNK), jnp.int32),
            pltpu.VMEM((cpw, _CHUNK), jnp.int32),
            pltpu.VMEM((cpw, _CHUNK), jnp.float32),
            pltpu.VMEM((_NPAD,), jnp.float32),
            pltpu.VMEM((_L,), jnp.int32),
        ],
    )
    return f(a1, a2, srcp, dstp)


def _agg_call(ft, av0, src0, dst0, av1, src1, dst1, nseg0, nseg1, srows):
    n, d = ft.shape
    rows_pt = _NPAD // _NS
    npair = srows // 2
    assert srows % 2 == 0
    mesh = plsc.VectorSubcoreMesh(core_axis_name="c", subcore_axis_name="s")

    def body(ft_hbm, a0_hbm, s0_hbm, d0_hbm, a1_hbm, s1_hbm, d1_hbm,
             part_hbm,
             src_v, dst_v, a_v, rows_a, rows_b, gs_a, gs_b, ss_a, ss_b,
             acc_sh):
        c = lax.axis_index("c")
        s = lax.axis_index("s")

        def zero_row(i, carry):
            for j in range(d // _L):
                rows_a[i, pl.ds(j * _L, _L)] = jnp.zeros((_L,), jnp.float32)
            return carry
        lax.fori_loop(0, _CHUNK, zero_row, 0)
        row0 = s * rows_pt
        for t in range(rows_pt // _CHUNK):
            pltpu.sync_copy(rows_a, acc_sh.at[pl.ds(row0 + t * _CHUNK, _CHUNK)])

        plsc.subcore_barrier()

        def issue_g(r, buf, sem):
            pltpu.async_copy(ft_hbm.at[src_v.at[r]], buf, sem)

        def drain_g(buf, sem):
            pltpu.make_async_copy(ft_hbm.at[src_v.at[0]], buf, sem).wait()

        def issue_s(r, buf, sem):
            pltpu.async_copy(buf, acc_sh.at[dst_v.at[r]], sem, add=True)

        def drain_s(buf, sem):
            pltpu.make_async_copy(buf, acc_sh.at[dst_v.at[0]], sem).wait()

        def scale(r, buf):
            def one(e, inner):
                av = plsc.load_gather(
                    a_v, [jnp.full((_L,), r, jnp.int32),
                          jnp.full((_L,), e, jnp.int32)])
                for j in range(d // _L):
                    buf[e, pl.ds(j * _L, _L)] = buf[e, pl.ds(j * _L, _L)] * av
                return inner
            lax.fori_loop(0, _CHUNK, one, 0)

        def run(a_hbm, src_hbm, dst_hbm, nseg):
            for seg in range(nseg):
                pltpu.sync_copy(src_hbm.at[s, seg], src_v)
                pltpu.sync_copy(dst_hbm.at[s, seg], dst_v)
                pltpu.sync_copy(a_hbm.at[s, seg], a_v)
                issue_g(0, rows_a, gs_a)

                def pair(i, carry):
                    p = 2 * i
                    drain_g(rows_a, gs_a)
                    scale(p, rows_a)

                    @pl.when(i > 0)
                    def _():
                        drain_s(rows_b, ss_b)
                    issue_g(p + 1, rows_b, gs_b)
                    issue_s(p, rows_a, ss_a)

                    drain_g(rows_b, gs_b)
                    scale(p + 1, rows_b)
                    drain_s(rows_a, ss_a)

                    @pl.when(i + 1 < npair)
                    def _():
                        issue_g(p + 2, rows_a, gs_a)
                    issue_s(p + 1, rows_b, ss_b)
                    return carry
                lax.fori_loop(0, npair, pair, 0)
                drain_s(rows_b, ss_b)

        @pl.when(c == 0)
        def _():
            run(a0_hbm, s0_hbm, d0_hbm, nseg0)

        @pl.when(c == 1)
        def _():
            run(a1_hbm, s1_hbm, d1_hbm, nseg1)

        plsc.subcore_barrier()

        pltpu.sync_copy(acc_sh.at[pl.ds(row0, rows_pt)],
                        part_hbm.at[c, pl.ds(row0, rows_pt)])

    f = pl.kernel(
        body,
        out_type=[
            jax.ShapeDtypeStruct((_NC, _NPAD, d), jnp.float32),
        ],
        mesh=mesh,
        compiler_params=_SC_PARAMS,
        scratch_types=[
            pltpu.VMEM((srows, _CHUNK), jnp.int32),
            pltpu.VMEM((srows, _CHUNK), jnp.int32),
            pltpu.VMEM((srows, _CHUNK), jnp.float32),
            pltpu.VMEM((_CHUNK, d), jnp.float32),
            pltpu.VMEM((_CHUNK, d), jnp.float32),
            pltpu.SemaphoreType.DMA,
            pltpu.SemaphoreType.DMA,
            pltpu.SemaphoreType.DMA,
            pltpu.SemaphoreType.DMA,
            pltpu.VMEM_SHARED((_NPAD, d), jnp.float32),
        ],
    )
    return f(ft, av0, src0, dst0, av1, src1, dst1)[0]


def _epilogue_call(part, zpart, cnt, n, d):
    blk = 1000

    def body(p_ref, z_ref, c_ref, out_ref, num_ref):
        ps = p_ref[0] + p_ref[1]
        z = jnp.sum(z_ref[...], axis=0)
        out_ref[...] = ps / z

        @pl.when(pl.program_id(0) == 0)
        def _():
            num_ref[0, 0] = jnp.sum(c_ref[...])

    return pl.pallas_call(
        body,
        grid=(n // blk,),
        in_specs=[
            pl.BlockSpec((_NC, blk, d), lambda i: (0, i, 0)),
            pl.BlockSpec((_NW, blk, 1), lambda i: (0, i, 0)),
            pl.BlockSpec((_NW, 1, _L), lambda i: (0, 0, 0)),
        ],
        out_specs=[
            pl.BlockSpec((blk, d), lambda i: (i, 0)),
            pl.BlockSpec(memory_space=pltpu.SMEM),
        ],
        out_shape=[
            jax.ShapeDtypeStruct((n, d), jnp.float32),
            jax.ShapeDtypeStruct((1, 1), jnp.int32),
        ],
    )(part, zpart, cnt)


def kernel(x, edge_index, W, b, attn_l, attn_r, bias_l0):
    n, d = x.shape
    e_total = edge_index.shape[1]
    cpw = -(-e_total // (_NW * _CHUNK))
    e_pad = _NW * cpw * _CHUNK
    srows = 24
    nseg0, nseg1 = 3, 4
    e_pad2 = _NS * (nseg0 + nseg1) * srows * _CHUNK
    ei = edge_index.astype(jnp.int32)
    pad = e_pad2 - e_total
    src_f = jnp.concatenate([ei[0], jnp.zeros((pad,), jnp.int32)])
    dst_f = jnp.concatenate([ei[1], jnp.zeros((pad,), jnp.int32)])
    src = src_f[:e_pad].reshape(_NW, cpw, _CHUNK)
    dst = dst_f[:e_pad].reshape(_NW, cpw, _CHUNK)

    ft, a1, a2 = _prologue_call(x, W, b.reshape(1, d), attn_l, attn_r,
                                bias_l0)
    av, zpart, cnt = _gate_call(a1.reshape(n), a2.reshape(n), src, dst,
                                cpw, e_total)
    e0 = _NS * nseg0 * srows * _CHUNK
    av_f = jnp.concatenate([av.reshape(e_pad),
                            jnp.zeros((e_pad2 - e_pad,), jnp.float32)])
    c0 = (_NS, nseg0, srows, _CHUNK)
    c1 = (_NS, nseg1, srows, _CHUNK)
    part = _agg_call(ft,
                     av_f[:e0].reshape(c0), src_f[:e0].reshape(c0),
                     dst_f[:e0].reshape(c0),
                     av_f[e0:].reshape(c1), src_f[e0:].reshape(c1),
                     dst_f[e0:].reshape(c1),
                     nseg0, nseg1, srows)
    out, num = _epilogue_call(part, zpart.reshape(_NW, _NPAD, 1), cnt, n, d)
    return out, num[0, 0]


# speedup vs baseline: 1.0555x; 1.0555x over previous
import jax
import jax.numpy as jnp
from jax import lax
from jax.experimental import pallas as pl
from jax.experimental.pallas import tpu as pltpu
from jax.experimental.pallas import tpu_sc as plsc

_BETA = 0.66
_GAMMA = -0.1
_ZETA = 1.1

_NC = 2
_NS = 16
_NW = _NC * _NS
_L = 16
_CHUNK = 128
_NPAD = 10240

_SC_PARAMS = pltpu.CompilerParams(
    needs_layout_passes=False, use_tc_tiling_on_sc=False)


def _prologue_call(x, W, b2, attn_l, attn_r, bias_l0):
    n, d = x.shape
    blk = 1000

    def body(x_ref, w_ref, b_ref, al_ref, ar_ref, bias_ref,
             ft_ref, a1_ref, a2_ref):
        ft = lax.dot_general(x_ref[...], w_ref[...], (((1,), (1,)), ((), ())),
                             preferred_element_type=jnp.float32)
        ft = ft + b_ref[...]
        ft_ref[...] = ft
        a1_ref[...] = jnp.sum(ft * al_ref[...], axis=1, keepdims=True)
        a2_ref[...] = jnp.sum(ft * ar_ref[...], axis=1, keepdims=True) + bias_ref[0]

    return pl.pallas_call(
        body,
        grid=(n // blk,),
        in_specs=[
            pl.BlockSpec((blk, d), lambda i: (i, 0)),
            pl.BlockSpec((d, d), lambda i: (0, 0)),
            pl.BlockSpec((1, d), lambda i: (0, 0)),
            pl.BlockSpec((1, d), lambda i: (0, 0)),
            pl.BlockSpec((1, d), lambda i: (0, 0)),
            pl.BlockSpec(memory_space=pltpu.SMEM),
        ],
        out_specs=[
            pl.BlockSpec((blk, d), lambda i: (i, 0)),
            pl.BlockSpec((blk, 1), lambda i: (i, 0)),
            pl.BlockSpec((blk, 1), lambda i: (i, 0)),
        ],
        out_shape=[
            jax.ShapeDtypeStruct((n, d), jnp.float32),
            jax.ShapeDtypeStruct((n, 1), jnp.float32),
            jax.ShapeDtypeStruct((n, 1), jnp.float32),
        ],
    )(x, W, b2, attn_l, attn_r, bias_l0)


def _gate_call(a1, a2, srcp, dstp, cpw, n_edges):
    n = a1.shape[0]
    rows_pt = _NPAD // _NS
    mesh = plsc.VectorSubcoreMesh(core_axis_name="c", subcore_axis_name="s")

    def body(a1_hbm, a2_hbm, src_hbm, dst_hbm, a_hbm, z_hbm, cnt_hbm,
             a1_v, a2_v, src_v, dst_v, a_v, z_v, cnt_v):
        c = lax.axis_index("c")
        s = lax.axis_index("s")
        wid = c * _NS + s

        def zero_z(i, carry):
            z_v[pl.ds(i * _L, _L)] = jnp.zeros((_L,), jnp.float32)
            return carry
        lax.fori_loop(0, _NPAD // _L, zero_z, 0)

        pltpu.sync_copy(a1_hbm, a1_v)
        pltpu.sync_copy(a2_hbm, a2_v)
        pltpu.sync_copy(src_hbm.at[wid], src_v)
        pltpu.sync_copy(dst_hbm.at[wid], dst_v)

        base = wid * (cpw * _CHUNK)
        lanes = lax.iota(jnp.int32, _L)

        def gate_row(r, cnt):
            for cc in range(_CHUNK // _L):
                sv = src_v[r, pl.ds(cc * _L, _L)]
                dv = dst_v[r, pl.ds(cc * _L, _L)]
                a1g = plsc.load_gather(a1_v, [sv])
                a2g = plsc.load_gather(a2_v, [dv])
                logit = a1g + a2g
                sg = 1.0 / (1.0 + jnp.exp(logit * (-1.0 / _BETA)))
                ab = sg * (_ZETA - _GAMMA) + _GAMMA
                a = jnp.clip(ab, 0.0, 1.0)
                a = jnp.where(sv == dv, 1.0, a)
                gid = base + r * _CHUNK + cc * _L + lanes
                a = jnp.where(gid < n_edges, a, 0.0)
                a_v[r, pl.ds(cc * _L, _L)] = a
                plsc.addupdate_scatter(z_v, [dv], a)
                cnt = cnt + jnp.where(a > 0.0, 1, 0).astype(jnp.int32)
            return cnt

        cnt = lax.fori_loop(0, cpw, gate_row, jnp.zeros((_L,), jnp.int32))
        pltpu.sync_copy(a_v, a_hbm.at[wid])
        cnt_v[...] = cnt
        pltpu.sync_copy(cnt_v, cnt_hbm.at[wid, 0])
        pltpu.sync_copy(z_v, z_hbm.at[wid, 0])

    f = pl.kernel(
        body,
        out_type=[
            jax.ShapeDtypeStruct((_NW, cpw, _CHUNK), jnp.float32),
            jax.ShapeDtypeStruct((_NW, 1, _NPAD), jnp.float32),
            jax.ShapeDtypeStruct((_NW, 1, _L), jnp.int32),
        ],
        mesh=mesh,
        compiler_params=_SC_PARAMS,
        scratch_types=[
            pltpu.VMEM((n,), jnp.float32),
            pltpu.VMEM((n,), jnp.float32),
            pltpu.VMEM((cpw, _CHUNK), jnp.int32),
            pltpu.VMEM((cpw, _CHUNK), jnp.int32),
            pltpu.VMEM((cpw, _CHUNK), jnp.float32),
            pltpu.VMEM((_NPAD,), jnp.float32),
            pltpu.VMEM((_L,), jnp.int32),
        ],
    )
    return f(a1, a2, srcp, dstp)


def _agg_call(ft, av_hbm, srcp, dstp, nseg, srows):
    n, d = ft.shape
    rows_pt = _NPAD // _NS
    npair = srows // 2
    assert srows % 2 == 0
    mesh = plsc.VectorSubcoreMesh(core_axis_name="c", subcore_axis_name="s")

    def body(ft_hbm, a_hbm, src_hbm, dst_hbm, part_hbm,
             src_v, dst_v, a_v, rows_a, rows_b, gs_a, gs_b, ss_a, ss_b,
             acc_sh):
        c = lax.axis_index("c")
        s = lax.axis_index("s")
        wid = c * _NS + s

        def zero_row(i, carry):
            for j in range(d // _L):
                rows_a[i, pl.ds(j * _L, _L)] = jnp.zeros((_L,), jnp.float32)
            return carry
        lax.fori_loop(0, _CHUNK, zero_row, 0)
        row0 = s * rows_pt
        for t in range(rows_pt // _CHUNK):
            pltpu.sync_copy(rows_a, acc_sh.at[pl.ds(row0 + t * _CHUNK, _CHUNK)])

        plsc.subcore_barrier()

        def issue_g(r, buf, sem):
            pltpu.async_copy(ft_hbm.at[src_v.at[r]], buf, sem)

        def drain_g(buf, sem):
            pltpu.make_async_copy(ft_hbm.at[src_v.at[0]], buf, sem).wait()

        def issue_s(r, buf, sem):
            pltpu.async_copy(buf, acc_sh.at[dst_v.at[r]], sem, add=True)

        def drain_s(buf, sem):
            pltpu.make_async_copy(buf, acc_sh.at[dst_v.at[0]], sem).wait()

        def scale(r, buf):
            def one(e, inner):
                av = plsc.load_gather(
                    a_v, [jnp.full((_L,), r, jnp.int32),
                          jnp.full((_L,), e, jnp.int32)])
                for j in range(d // _L):
                    buf[e, pl.ds(j * _L, _L)] = buf[e, pl.ds(j * _L, _L)] * av
                return inner
            lax.fori_loop(0, _CHUNK, one, 0)

        for seg in range(nseg):
            pltpu.sync_copy(src_hbm.at[wid, seg], src_v)
            pltpu.sync_copy(dst_hbm.at[wid, seg], dst_v)
            pltpu.sync_copy(a_hbm.at[wid, seg], a_v)
            issue_g(0, rows_a, gs_a)

            def pair(i, carry):
                p = 2 * i
                drain_g(rows_a, gs_a)
                scale(p, rows_a)

                @pl.when(i > 0)
                def _():
                    drain_s(rows_b, ss_b)
                issue_g(p + 1, rows_b, gs_b)
                issue_s(p, rows_a, ss_a)

                drain_g(rows_b, gs_b)
                scale(p + 1, rows_b)
                drain_s(rows_a, ss_a)

                @pl.when(i + 1 < npair)
                def _():
                    issue_g(p + 2, rows_a, gs_a)
                issue_s(p + 1, rows_b, ss_b)
                return carry
            lax.fori_loop(0, npair, pair, 0)
            drain_s(rows_b, ss_b)

        plsc.subcore_barrier()

        pltpu.sync_copy(acc_sh.at[pl.ds(row0, rows_pt)],
                        part_hbm.at[c, pl.ds(row0, rows_pt)])

    f = pl.kernel(
        body,
        out_type=[
            jax.ShapeDtypeStruct((_NC, _NPAD, d), jnp.float32),
        ],
        mesh=mesh,
        compiler_params=_SC_PARAMS,
        scratch_types=[
            pltpu.VMEM((srows, _CHUNK), jnp.int32),
            pltpu.VMEM((srows, _CHUNK), jnp.int32),
            pltpu.VMEM((srows, _CHUNK), jnp.float32),
            pltpu.VMEM((_CHUNK, d), jnp.float32),
            pltpu.VMEM((_CHUNK, d), jnp.float32),
            pltpu.SemaphoreType.DMA,
            pltpu.SemaphoreType.DMA,
            pltpu.SemaphoreType.DMA,
            pltpu.SemaphoreType.DMA,
            pltpu.VMEM_SHARED((_NPAD, d), jnp.float32),
        ],
    )
    return f(ft, av_hbm, srcp, dstp)[0]


def _epilogue_call(part, zpart, cnt, n, d):
    blk = 1000

    def body(p_ref, z_ref, c_ref, out_ref, num_ref):
        ps = p_ref[0] + p_ref[1]
        z = jnp.sum(z_ref[...], axis=0)
        out_ref[...] = ps / z

        @pl.when(pl.program_id(0) == 0)
        def _():
            num_ref[0, 0] = jnp.sum(c_ref[...])

    return pl.pallas_call(
        body,
        grid=(n // blk,),
        in_specs=[
            pl.BlockSpec((_NC, blk, d), lambda i: (0, i, 0)),
            pl.BlockSpec((_NW, blk, 1), lambda i: (0, i, 0)),
            pl.BlockSpec((_NW, 1, _L), lambda i: (0, 0, 0)),
        ],
        out_specs=[
            pl.BlockSpec((blk, d), lambda i: (i, 0)),
            pl.BlockSpec(memory_space=pltpu.SMEM),
        ],
        out_shape=[
            jax.ShapeDtypeStruct((n, d), jnp.float32),
            jax.ShapeDtypeStruct((1, 1), jnp.int32),
        ],
    )(part, zpart, cnt)


def kernel(x, edge_index, W, b, attn_l, attn_r, bias_l0):
    n, d = x.shape
    e_total = edge_index.shape[1]
    nseg, srows = 3, 28
    cpw = nseg * srows
    e_pad = _NW * cpw * _CHUNK
    ei = edge_index.astype(jnp.int32)
    pad = e_pad - e_total
    src = jnp.concatenate([ei[0], jnp.zeros((pad,), jnp.int32)])
    dst = jnp.concatenate([ei[1], jnp.zeros((pad,), jnp.int32)])
    src = src.reshape(_NW, cpw, _CHUNK)
    dst = dst.reshape(_NW, cpw, _CHUNK)

    ft, a1, a2 = _prologue_call(x, W, b.reshape(1, d), attn_l, attn_r,
                                bias_l0)
    av, zpart, cnt = _gate_call(a1.reshape(n), a2.reshape(n), src, dst,
                                cpw, e_total)
    seg4 = (_NW, nseg, srows, _CHUNK)
    part = _agg_call(ft, av.reshape(seg4), src.reshape(seg4),
                     dst.reshape(seg4), nseg, srows)
    out, num = _epilogue_call(part, zpart.reshape(_NW, _NPAD, 1), cnt, n, d)
    return out, num[0, 0]


# speedup vs baseline: 2.3243x; 2.2022x over previous
import jax
import jax.numpy as jnp
from jax import lax
from jax.experimental import pallas as pl
from jax.experimental.pallas import tpu as pltpu
from jax.experimental.pallas import tpu_sc as plsc

_BETA = 0.66
_GAMMA = -0.1
_ZETA = 1.1

_NC = 2
_NS = 16
_NW = _NC * _NS
_L = 16
_CHUNK = 128
_NPAD = 10240

_SC_PARAMS = pltpu.CompilerParams(
    needs_layout_passes=False, use_tc_tiling_on_sc=False)


def _prologue_call(x, W, b2, attn_l, attn_r, bias_l0):
    n, d = x.shape
    blk = 1000

    def body(x_ref, w_ref, b_ref, al_ref, ar_ref, bias_ref,
             ft_ref, a1_ref, a2_ref):
        ft = lax.dot_general(x_ref[...], w_ref[...], (((1,), (1,)), ((), ())),
                             preferred_element_type=jnp.float32)
        ft = ft + b_ref[...]
        ft_ref[...] = ft
        a1_ref[...] = jnp.sum(ft * al_ref[...], axis=1, keepdims=True)
        a2_ref[...] = jnp.sum(ft * ar_ref[...], axis=1, keepdims=True) + bias_ref[0]

    return pl.pallas_call(
        body,
        grid=(n // blk,),
        in_specs=[
            pl.BlockSpec((blk, d), lambda i: (i, 0)),
            pl.BlockSpec((d, d), lambda i: (0, 0)),
            pl.BlockSpec((1, d), lambda i: (0, 0)),
            pl.BlockSpec((1, d), lambda i: (0, 0)),
            pl.BlockSpec((1, d), lambda i: (0, 0)),
            pl.BlockSpec(memory_space=pltpu.SMEM),
        ],
        out_specs=[
            pl.BlockSpec((blk, d), lambda i: (i, 0)),
            pl.BlockSpec((blk, 1), lambda i: (i, 0)),
            pl.BlockSpec((blk, 1), lambda i: (i, 0)),
        ],
        out_shape=[
            jax.ShapeDtypeStruct((n, d), jnp.float32),
            jax.ShapeDtypeStruct((n, 1), jnp.float32),
            jax.ShapeDtypeStruct((n, 1), jnp.float32),
        ],
    )(x, W, b2, attn_l, attn_r, bias_l0)


def _gate_call(a1, a2, srcp, dstp, cpw, n_edges):
    n = a1.shape[0]
    rows_pt = _NPAD // _NS
    mesh = plsc.VectorSubcoreMesh(core_axis_name="c", subcore_axis_name="s")

    def body(a1_hbm, a2_hbm, src_hbm, dst_hbm, a_hbm, z_hbm, cnt_hbm,
             a1_v, a2_v, src_v, dst_v, a_v, z_v, cnt_v):
        c = lax.axis_index("c")
        s = lax.axis_index("s")
        wid = c * _NS + s

        def zero_z(i, carry):
            z_v[pl.ds(i * _L, _L)] = jnp.zeros((_L,), jnp.float32)
            return carry
        lax.fori_loop(0, _NPAD // _L, zero_z, 0)

        pltpu.sync_copy(a1_hbm, a1_v)
        pltpu.sync_copy(a2_hbm, a2_v)
        pltpu.sync_copy(src_hbm.at[wid], src_v)
        pltpu.sync_copy(dst_hbm.at[wid], dst_v)

        base = wid * (cpw * _CHUNK)
        lanes = lax.iota(jnp.int32, _L)

        def gate_row(r, cnt):
            for cc in range(_CHUNK // _L):
                sv = src_v[r, pl.ds(cc * _L, _L)]
                dv = dst_v[r, pl.ds(cc * _L, _L)]
                a1g = plsc.load_gather(a1_v, [sv])
                a2g = plsc.load_gather(a2_v, [dv])
                logit = a1g + a2g
                sg = 1.0 / (1.0 + jnp.exp(logit * (-1.0 / _BETA)))
                ab = sg * (_ZETA - _GAMMA) + _GAMMA
                a = jnp.clip(ab, 0.0, 1.0)
                a = jnp.where(sv == dv, 1.0, a)
                gid = base + r * _CHUNK + cc * _L + lanes
                a = jnp.where(gid < n_edges, a, 0.0)
                a_v[r, pl.ds(cc * _L, _L)] = a
                plsc.addupdate_scatter(z_v, [dv], a)
                cnt = cnt + jnp.where(a > 0.0, 1, 0).astype(jnp.int32)
            return cnt

        cnt = lax.fori_loop(0, cpw, gate_row, jnp.zeros((_L,), jnp.int32))
        pltpu.sync_copy(a_v, a_hbm.at[wid])
        cnt_v[...] = cnt
        pltpu.sync_copy(cnt_v, cnt_hbm.at[wid, 0])
        pltpu.sync_copy(z_v, z_hbm.at[wid, 0])

    f = pl.kernel(
        body,
        out_type=[
            jax.ShapeDtypeStruct((_NW, cpw, _CHUNK), jnp.float32),
            jax.ShapeDtypeStruct((_NW, 1, _NPAD), jnp.float32),
            jax.ShapeDtypeStruct((_NW, 1, _L), jnp.int32),
        ],
        mesh=mesh,
        compiler_params=_SC_PARAMS,
        scratch_types=[
            pltpu.VMEM((n,), jnp.float32),
            pltpu.VMEM((n,), jnp.float32),
            pltpu.VMEM((cpw, _CHUNK), jnp.int32),
            pltpu.VMEM((cpw, _CHUNK), jnp.int32),
            pltpu.VMEM((cpw, _CHUNK), jnp.float32),
            pltpu.VMEM((_NPAD,), jnp.float32),
            pltpu.VMEM((_L,), jnp.int32),
        ],
    )
    return f(a1, a2, srcp, dstp)


def _agg_call(ft, av_hbm, srcp, dstp, nseg, srows):
    n, d = ft.shape
    rows_pt = _NPAD // _NS
    npair = srows // 2
    mesh = plsc.VectorSubcoreMesh(core_axis_name="c", subcore_axis_name="s")

    def body(ft_hbm, a_hbm, src_hbm, dst_hbm, part_hbm,
             src_v, dst_v, a_v, rows_a, rows_b, gs_a, gs_b, ss_a, ss_b,
             acc_sh):
        c = lax.axis_index("c")
        s = lax.axis_index("s")
        wid = c * _NS + s

        def zero_row(i, carry):
            for j in range(d // _L):
                rows_a[i, pl.ds(j * _L, _L)] = jnp.zeros((_L,), jnp.float32)
            return carry
        lax.fori_loop(0, _CHUNK, zero_row, 0)
        row0 = s * rows_pt
        for t in range(rows_pt // _CHUNK):
            pltpu.sync_copy(rows_a, acc_sh.at[pl.ds(row0 + t * _CHUNK, _CHUNK)])

        plsc.subcore_barrier()

        def issue_g(r, buf, sem):
            pltpu.async_copy(ft_hbm.at[src_v.at[r]], buf, sem)

        def drain_g(buf, sem):
            pltpu.make_async_copy(ft_hbm.at[src_v.at[0]], buf, sem).wait()

        def issue_s(r, buf, sem):
            pltpu.async_copy(buf, acc_sh.at[dst_v.at[r]], sem, add=True)

        def drain_s(buf, sem):
            pltpu.make_async_copy(buf, acc_sh.at[dst_v.at[0]], sem).wait()

        def scale(r, buf):
            def one(e, inner):
                av = plsc.load_gather(
                    a_v, [jnp.full((_L,), r, jnp.int32),
                          jnp.full((_L,), e, jnp.int32)])
                for j in range(d // _L):
                    buf[e, pl.ds(j * _L, _L)] = buf[e, pl.ds(j * _L, _L)] * av
                return inner
            lax.fori_loop(0, _CHUNK, one, 0)

        for seg in range(nseg):
            pltpu.sync_copy(src_hbm.at[wid, seg], src_v)
            pltpu.sync_copy(dst_hbm.at[wid, seg], dst_v)
            pltpu.sync_copy(a_hbm.at[wid, seg], a_v)
            issue_g(0, rows_a, gs_a)

            def pair(i, carry):
                p = 2 * i
                drain_g(rows_a, gs_a)
                scale(p, rows_a)

                @pl.when(i > 0)
                def _():
                    drain_s(rows_b, ss_b)
                issue_g(p + 1, rows_b, gs_b)
                issue_s(p, rows_a, ss_a)

                drain_g(rows_b, gs_b)
                scale(p + 1, rows_b)
                drain_s(rows_a, ss_a)

                @pl.when(i + 1 < npair + (srows % 2))
                def _():
                    issue_g(p + 2, rows_a, gs_a)
                issue_s(p + 1, rows_b, ss_b)
                return carry
            lax.fori_loop(0, npair, pair, 0)

            if srows % 2:
                drain_g(rows_a, gs_a)
                scale(srows - 1, rows_a)
                drain_s(rows_b, ss_b)
                issue_s(srows - 1, rows_a, ss_a)
                drain_s(rows_a, ss_a)
            else:
                drain_s(rows_b, ss_b)

        plsc.subcore_barrier()

        pltpu.sync_copy(acc_sh.at[pl.ds(row0, rows_pt)],
                        part_hbm.at[c, pl.ds(row0, rows_pt)])

    f = pl.kernel(
        body,
        out_type=[
            jax.ShapeDtypeStruct((_NC, _NPAD, d), jnp.float32),
        ],
        mesh=mesh,
        compiler_params=_SC_PARAMS,
        scratch_types=[
            pltpu.VMEM((srows, _CHUNK), jnp.int32),
            pltpu.VMEM((srows, _CHUNK), jnp.int32),
            pltpu.VMEM((srows, _CHUNK), jnp.float32),
            pltpu.VMEM((_CHUNK, d), jnp.float32),
            pltpu.VMEM((_CHUNK, d), jnp.float32),
            pltpu.SemaphoreType.DMA,
            pltpu.SemaphoreType.DMA,
            pltpu.SemaphoreType.DMA,
            pltpu.SemaphoreType.DMA,
            pltpu.VMEM_SHARED((_NPAD, d), jnp.float32),
        ],
    )
    return f(ft, av_hbm, srcp, dstp)[0]


def _epilogue_call(part, zpart, cnt, n, d):
    blk = 1000

    def body(p_ref, z_ref, c_ref, out_ref, num_ref):
        ps = p_ref[0] + p_ref[1]
        z = jnp.sum(z_ref[...], axis=0)
        out_ref[...] = ps / z

        @pl.when(pl.program_id(0) == 0)
        def _():
            num_ref[0, 0] = jnp.sum(c_ref[...])

    return pl.pallas_call(
        body,
        grid=(n // blk,),
        in_specs=[
            pl.BlockSpec((_NC, blk, d), lambda i: (0, i, 0)),
            pl.BlockSpec((_NW, blk, 1), lambda i: (0, i, 0)),
            pl.BlockSpec((_NW, 1, _L), lambda i: (0, 0, 0)),
        ],
        out_specs=[
            pl.BlockSpec((blk, d), lambda i: (i, 0)),
            pl.BlockSpec(memory_space=pltpu.SMEM),
        ],
        out_shape=[
            jax.ShapeDtypeStruct((n, d), jnp.float32),
            jax.ShapeDtypeStruct((1, 1), jnp.int32),
        ],
    )(part, zpart, cnt)


def kernel(x, edge_index, W, b, attn_l, attn_r, bias_l0):
    n, d = x.shape
    e_total = edge_index.shape[1]
    nseg, srows = 3, 27
    cpw = nseg * srows
    e_pad = _NW * cpw * _CHUNK
    ei = edge_index.astype(jnp.int32)
    pad = e_pad - e_total
    src = jnp.concatenate([ei[0], jnp.zeros((pad,), jnp.int32)])
    dst = jnp.concatenate([ei[1], jnp.zeros((pad,), jnp.int32)])
    src = src.reshape(_NW, cpw, _CHUNK)
    dst = dst.reshape(_NW, cpw, _CHUNK)

    ft, a1, a2 = _prologue_call(x, W, b.reshape(1, d), attn_l, attn_r,
                                bias_l0)
    av, zpart, cnt = _gate_call(a1.reshape(n), a2.reshape(n), src, dst,
                                cpw, e_total)
    seg4 = (_NW, nseg, srows, _CHUNK)
    part = _agg_call(ft, av.reshape(seg4), src.reshape(seg4),
                     dst.reshape(seg4), nseg, srows)
    out, num = _epilogue_call(part, zpart.reshape(_NW, _NPAD, 1), cnt, n, d)
    return out, num[0, 0]
